# Initial kernel scaffold; baseline (speedup 1.0000x reference)
import functools

import jax
import jax.numpy as jnp
from jax import lax
from jax.experimental import pallas as pl
from jax.experimental.pallas import tpu as pltpu
from jax.experimental.pallas import tpu_sc as plsc

NUM_NODES = 1100000
EMBED_DIM = 64
NUM_EDGES = 1000000

NC = 2
NS = 16
NW = NC * NS
L = 16

CHUNK = 512
SUB = 128
NSUB = CHUNK // SUB
B_PER_W = 31744
NCHUNKS = B_PER_W // CHUNK
E_PAD = B_PER_W * NW
GROUPS = CHUNK // L


def _sc_kernel_body(table, src_idx, dst_idx, out_hbm,
                    idx_s, idx_d, rows_s, rows_d, rbuf, obuf, sem):
    wid = lax.axis_index("s") * NC + lax.axis_index("c")
    row0 = wid * (B_PER_W // SUB)
    e0 = wid * B_PER_W
    iota = lax.iota(jnp.int32, L)
    tr_base = iota * L

    def chunk_body(i, _):
        r = row0 + i * NSUB
        pltpu.sync_copy(src_idx.at[pl.ds(r, NSUB)], idx_s)
        pltpu.sync_copy(dst_idx.at[pl.ds(r, NSUB)], idx_d)
        copies = []
        for j in range(NSUB):
            copies.append(pltpu.async_copy(
                table.at[idx_s.at[j]], rows_s.at[pl.ds(j * SUB, SUB)], sem))
            copies.append(pltpu.async_copy(
                table.at[idx_d.at[j]], rows_d.at[pl.ds(j * SUB, SUB)], sem))
        for c in copies:
            c.wait()

        def group_body(g, _):
            eb = g * L
            for e in range(L):
                er = eb + e
                p = rows_s[er, pl.ds(0, L)] * rows_d[er, pl.ds(0, L)]
                for k in range(1, EMBED_DIM // L):
                    p = p + rows_s[er, pl.ds(k * L, L)] * rows_d[er, pl.ds(k * L, L)]
                rbuf[pl.ds(e * L, L)] = p
            o = plsc.load_gather(rbuf, [tr_base])
            for l in range(1, L):
                o = o + plsc.load_gather(rbuf, [tr_base + l])
            obuf[pl.ds(eb, L)] = o
            return ()

        lax.fori_loop(0, GROUPS, group_body, (), unroll=False)
        pltpu.sync_copy(obuf, out_hbm.at[pl.ds(e0 + i * CHUNK, CHUNK)])
        return ()

    lax.fori_loop(0, NCHUNKS, chunk_body, (), unroll=False)


@functools.partial(
    pl.kernel,
    mesh=plsc.VectorSubcoreMesh(core_axis_name="c", subcore_axis_name="s"),
    out_type=jax.ShapeDtypeStruct((E_PAD,), jnp.float32),
    scratch_types=[
        pltpu.VMEM((NSUB, SUB), jnp.int32),
        pltpu.VMEM((NSUB, SUB), jnp.int32),
        pltpu.VMEM((CHUNK, EMBED_DIM), jnp.float32),
        pltpu.VMEM((CHUNK, EMBED_DIM), jnp.float32),
        pltpu.VMEM((L * L,), jnp.float32),
        pltpu.VMEM((CHUNK,), jnp.float32),
        pltpu.SemaphoreType.DMA,
    ],
)
def _score_edges(table, src_idx, dst_idx, out_hbm, *scratch):
    _sc_kernel_body(table, src_idx, dst_idx, out_hbm, *scratch)


def kernel(embedding, edge_index):
    pad = E_PAD - NUM_EDGES
    src = jnp.pad(edge_index[0], (0, pad)).reshape(E_PAD // SUB, SUB)
    dst = jnp.pad(edge_index[1], (0, pad)).reshape(E_PAD // SUB, SUB)
    scores = _score_edges(embedding, src, dst)
    return scores[:NUM_EDGES].reshape(NUM_EDGES, 1)


# speedup vs baseline: 1.0806x; 1.0806x over previous
import functools

import jax
import jax.numpy as jnp
from jax import lax
from jax.experimental import pallas as pl
from jax.experimental.pallas import tpu as pltpu
from jax.experimental.pallas import tpu_sc as plsc

NUM_NODES = 1100000
EMBED_DIM = 64
NUM_EDGES = 1000000

NC = 2
NS = 16
NW = NC * NS
L = 16

CHUNK = 512
SUB = 128
NSUB = CHUNK // SUB
B_PER_W = 31744
NCHUNKS = B_PER_W // CHUNK
E_PAD = B_PER_W * NW
GROUPS = CHUNK // L


def _sc_kernel_body(table, src_idx, dst_idx, out_hbm,
                    idx_s, idx_d, rows_s, rows_d, obuf, sem):
    wid = lax.axis_index("s") * NC + lax.axis_index("c")
    row0 = wid * (B_PER_W // SUB)
    e0 = wid * B_PER_W
    iota = lax.iota(jnp.int32, L)

    def chunk_body(i, _):
        r = row0 + i * NSUB
        pltpu.sync_copy(src_idx.at[pl.ds(r, NSUB)], idx_s)
        pltpu.sync_copy(dst_idx.at[pl.ds(r, NSUB)], idx_d)
        copies = []
        for j in range(NSUB):
            copies.append(pltpu.async_copy(
                table.at[idx_s.at[j]], rows_s.at[pl.ds(j * SUB, SUB)], sem))
            copies.append(pltpu.async_copy(
                table.at[idx_d.at[j]], rows_d.at[pl.ds(j * SUB, SUB)], sem))
        for c in copies:
            c.wait()

        def group_body(g, _):
            eb = g * L
            o = jnp.zeros((L,), jnp.float32)
            for e in range(L):
                er = eb + e
                p = rows_s[er, pl.ds(0, L)] * rows_d[er, pl.ds(0, L)]
                for k in range(1, EMBED_DIM // L):
                    p = p + rows_s[er, pl.ds(k * L, L)] * rows_d[er, pl.ds(k * L, L)]
                o = jnp.where(iota == e, jnp.sum(p), o)
            obuf[pl.ds(eb, L)] = o
            return ()

        lax.fori_loop(0, GROUPS, group_body, (), unroll=False)
        pltpu.sync_copy(obuf, out_hbm.at[pl.ds(e0 + i * CHUNK, CHUNK)])
        return ()

    lax.fori_loop(0, NCHUNKS, chunk_body, (), unroll=False)


@functools.partial(
    pl.kernel,
    mesh=plsc.VectorSubcoreMesh(core_axis_name="c", subcore_axis_name="s"),
    out_type=jax.ShapeDtypeStruct((E_PAD,), jnp.float32),
    compiler_params=pltpu.CompilerParams(
        needs_layout_passes=False, use_tc_tiling_on_sc=False),
    scratch_types=[
        pltpu.VMEM((NSUB, SUB), jnp.int32),
        pltpu.VMEM((NSUB, SUB), jnp.int32),
        pltpu.VMEM((CHUNK, EMBED_DIM), jnp.float32),
        pltpu.VMEM((CHUNK, EMBED_DIM), jnp.float32),
        pltpu.VMEM((CHUNK,), jnp.float32),
        pltpu.SemaphoreType.DMA,
    ],
)
def _score_edges(table, src_idx, dst_idx, out_hbm, *scratch):
    _sc_kernel_body(table, src_idx, dst_idx, out_hbm, *scratch)


def kernel(embedding, edge_index):
    pad = E_PAD - NUM_EDGES
    src = jnp.pad(edge_index[0], (0, pad)).reshape(E_PAD // SUB, SUB)
    dst = jnp.pad(edge_index[1], (0, pad)).reshape(E_PAD // SUB, SUB)
    scores = _score_edges(embedding, src, dst)
    return scores[:NUM_EDGES].reshape(NUM_EDGES, 1)


# speedup vs baseline: 1.4377x; 1.3305x over previous
import functools

import jax
import jax.numpy as jnp
from jax import lax
from jax.experimental import pallas as pl
from jax.experimental.pallas import tpu as pltpu
from jax.experimental.pallas import tpu_sc as plsc

NUM_NODES = 1100000
EMBED_DIM = 64
NUM_EDGES = 1000000

NC = 2
NS = 16
NW = NC * NS
L = 16

CHUNK = 384
SUB = 128
NSUB = CHUNK // SUB
B_PER_W = 31488
NCHUNKS = B_PER_W // CHUNK
E_PAD = B_PER_W * NW
GROUPS = CHUNK // L


def _sc_kernel_body(table, src_idx, dst_idx, out_hbm,
                    idx_s0, idx_d0, idx_s1, idx_d1,
                    rows_s0, rows_d0, rows_s1, rows_d1,
                    obuf, sem0, sem1):
    wid = lax.axis_index("s") * NC + lax.axis_index("c")
    row0 = wid * (B_PER_W // SUB)
    e0 = wid * B_PER_W
    iota = lax.iota(jnp.int32, L)
    last_lane = iota == (L - 1)

    def fire(c, idx_s, idx_d, rows_s, rows_d, sem):
        r = row0 + c * NSUB
        pltpu.sync_copy(src_idx.at[pl.ds(r, NSUB)], idx_s)
        pltpu.sync_copy(dst_idx.at[pl.ds(r, NSUB)], idx_d)
        for j in range(NSUB):
            pltpu.async_copy(
                table.at[idx_s.at[j]], rows_s.at[pl.ds(j * SUB, SUB)], sem)
            pltpu.async_copy(
                table.at[idx_d.at[j]], rows_d.at[pl.ds(j * SUB, SUB)], sem)

    def drain(rows_s, rows_d, sem):
        pltpu.make_async_copy(
            table.at[pl.ds(0, CHUNK)], rows_s, sem).wait()
        pltpu.make_async_copy(
            table.at[pl.ds(0, CHUNK)], rows_d, sem).wait()

    def compute(c, rows_s, rows_d):
        def group_body(g, _):
            eb = g * L
            for e in range(L):
                er = eb + e
                p = rows_s[er, pl.ds(0, L)] * rows_d[er, pl.ds(0, L)]
                for k in range(1, EMBED_DIM // L):
                    p = p + rows_s[er, pl.ds(k * L, L)] * rows_d[er, pl.ds(k * L, L)]
                cs = plsc.cumsum(p)
                plsc.store_compressed(obuf.at[pl.ds(er, L)], cs, mask=last_lane)
            return ()

        lax.fori_loop(0, GROUPS, group_body, (), unroll=False)
        pltpu.sync_copy(obuf.at[pl.ds(0, CHUNK)],
                        out_hbm.at[pl.ds(e0 + c * CHUNK, CHUNK)])

    bufs = ((idx_s0, idx_d0, rows_s0, rows_d0, sem0),
            (idx_s1, idx_d1, rows_s1, rows_d1, sem1))

    fire(0, *bufs[0])
    fire(1, *bufs[1])

    def chunk_pair(i2, _):
        for b in range(2):
            c = i2 * 2 + b
            idx_s, idx_d, rows_s, rows_d, sem = bufs[b]
            drain(rows_s, rows_d, sem)
            compute(c, rows_s, rows_d)

            @pl.when(c + 2 < NCHUNKS)
            def _():
                fire(c + 2, idx_s, idx_d, rows_s, rows_d, sem)
        return ()

    lax.fori_loop(0, NCHUNKS // 2, chunk_pair, (), unroll=False)


@functools.partial(
    pl.kernel,
    mesh=plsc.VectorSubcoreMesh(core_axis_name="c", subcore_axis_name="s"),
    out_type=jax.ShapeDtypeStruct((E_PAD,), jnp.float32),
    compiler_params=pltpu.CompilerParams(
        needs_layout_passes=False, use_tc_tiling_on_sc=False),
    scratch_types=[
        pltpu.VMEM((NSUB, SUB), jnp.int32),
        pltpu.VMEM((NSUB, SUB), jnp.int32),
        pltpu.VMEM((NSUB, SUB), jnp.int32),
        pltpu.VMEM((NSUB, SUB), jnp.int32),
        pltpu.VMEM((CHUNK, EMBED_DIM), jnp.float32),
        pltpu.VMEM((CHUNK, EMBED_DIM), jnp.float32),
        pltpu.VMEM((CHUNK, EMBED_DIM), jnp.float32),
        pltpu.VMEM((CHUNK, EMBED_DIM), jnp.float32),
        pltpu.VMEM((CHUNK + L,), jnp.float32),
        pltpu.SemaphoreType.DMA,
        pltpu.SemaphoreType.DMA,
    ],
)
def _score_edges(table, src_idx, dst_idx, out_hbm, *scratch):
    _sc_kernel_body(table, src_idx, dst_idx, out_hbm, *scratch)


def kernel(embedding, edge_index):
    pad = E_PAD - NUM_EDGES
    src = jnp.pad(edge_index[0], (0, pad)).reshape(E_PAD // SUB, SUB)
    dst = jnp.pad(edge_index[1], (0, pad)).reshape(E_PAD // SUB, SUB)
    scores = _score_edges(embedding, src, dst)
    return scores[:NUM_EDGES].reshape(NUM_EDGES, 1)


# speedup vs baseline: 1.5071x; 1.0483x over previous
import functools

import jax
import jax.numpy as jnp
from jax import lax
from jax.experimental import pallas as pl
from jax.experimental.pallas import tpu as pltpu
from jax.experimental.pallas import tpu_sc as plsc

NUM_NODES = 1100000
EMBED_DIM = 64
NUM_EDGES = 1000000

NC = 2
NS = 16
NW = NC * NS
L = 16

CHUNK = 384
SUB = 128
NSUB = CHUNK // SUB
B_PER_W = 31488
NCHUNKS = B_PER_W // CHUNK
E_PAD = B_PER_W * NW
GROUPS = CHUNK // L


def _sc_kernel_body(table, src_idx, dst_idx, out_hbm,
                    idx_s0, idx_d0, idx_s1, idx_d1,
                    rows_s0, rows_d0, rows_s1, rows_d1,
                    obuf0, obuf1, sem0, sem1, isem0, isem1, osem0, osem1):
    wid = lax.axis_index("s") * NC + lax.axis_index("c")
    e0 = wid * B_PER_W
    iota = lax.iota(jnp.int32, L)
    last_lane = iota == (L - 1)

    def idx_copy(c, idx_s, idx_d, isem):
        off = e0 + c * CHUNK
        pltpu.async_copy(src_idx.at[pl.ds(off, CHUNK)], idx_s, isem)
        pltpu.async_copy(dst_idx.at[pl.ds(off, CHUNK)], idx_d, isem)

    def idx_wait(idx_s, idx_d, isem):
        pltpu.make_async_copy(src_idx.at[pl.ds(0, CHUNK)], idx_s, isem).wait()
        pltpu.make_async_copy(dst_idx.at[pl.ds(0, CHUNK)], idx_d, isem).wait()

    def fire(idx_s, idx_d, rows_s, rows_d, sem):
        for j in range(NSUB):
            pltpu.async_copy(
                table.at[idx_s.at[pl.ds(j * SUB, SUB)]],
                rows_s.at[pl.ds(j * SUB, SUB)], sem)
            pltpu.async_copy(
                table.at[idx_d.at[pl.ds(j * SUB, SUB)]],
                rows_d.at[pl.ds(j * SUB, SUB)], sem)

    def drain(rows_s, rows_d, sem):
        pltpu.make_async_copy(table.at[pl.ds(0, CHUNK)], rows_s, sem).wait()
        pltpu.make_async_copy(table.at[pl.ds(0, CHUNK)], rows_d, sem).wait()

    def compute(rows_s, rows_d, obuf):
        def group_body(g, _):
            eb = g * L
            for e in range(L):
                er = eb + e
                p = rows_s[er, pl.ds(0, L)] * rows_d[er, pl.ds(0, L)]
                for k in range(1, EMBED_DIM // L):
                    p = p + rows_s[er, pl.ds(k * L, L)] * rows_d[er, pl.ds(k * L, L)]
                cs = plsc.cumsum(p)
                plsc.store_compressed(obuf.at[pl.ds(er, L)], cs, mask=last_lane)
            return ()

        lax.fori_loop(0, GROUPS, group_body, (), unroll=False)

    bufs = ((idx_s0, idx_d0, rows_s0, rows_d0, obuf0, sem0, isem0, osem0),
            (idx_s1, idx_d1, rows_s1, rows_d1, obuf1, sem1, isem1, osem1))

    for b in range(2):
        idx_s, idx_d, rows_s, rows_d, obuf, sem, isem, osem = bufs[b]
        idx_copy(b, idx_s, idx_d, isem)
        idx_wait(idx_s, idx_d, isem)
        fire(idx_s, idx_d, rows_s, rows_d, sem)

    def chunk_pair(i2, _):
        for b in range(2):
            c = i2 * 2 + b
            idx_s, idx_d, rows_s, rows_d, obuf, sem, isem, osem = bufs[b]

            @pl.when(c >= 2)
            def _():
                pltpu.make_async_copy(
                    obuf.at[pl.ds(0, CHUNK)],
                    out_hbm.at[pl.ds(0, CHUNK)], osem).wait()

            drain(rows_s, rows_d, sem)

            @pl.when(c + 2 < NCHUNKS)
            def _():
                idx_copy(c + 2, idx_s, idx_d, isem)

            compute(rows_s, rows_d, obuf)
            pltpu.async_copy(obuf.at[pl.ds(0, CHUNK)],
                             out_hbm.at[pl.ds(e0 + c * CHUNK, CHUNK)], osem)

            @pl.when(c + 2 < NCHUNKS)
            def _():
                idx_wait(idx_s, idx_d, isem)
                fire(idx_s, idx_d, rows_s, rows_d, sem)
        return ()

    lax.fori_loop(0, NCHUNKS // 2, chunk_pair, (), unroll=False)

    for b in range(2):
        _, _, _, _, obuf, _, _, osem = bufs[b]
        pltpu.make_async_copy(obuf.at[pl.ds(0, CHUNK)],
                              out_hbm.at[pl.ds(0, CHUNK)], osem).wait()


@functools.partial(
    pl.kernel,
    mesh=plsc.VectorSubcoreMesh(core_axis_name="c", subcore_axis_name="s"),
    out_type=jax.ShapeDtypeStruct((E_PAD,), jnp.float32),
    compiler_params=pltpu.CompilerParams(
        needs_layout_passes=False, use_tc_tiling_on_sc=False),
    scratch_types=[
        pltpu.VMEM((CHUNK,), jnp.int32),
        pltpu.VMEM((CHUNK,), jnp.int32),
        pltpu.VMEM((CHUNK,), jnp.int32),
        pltpu.VMEM((CHUNK,), jnp.int32),
        pltpu.VMEM((CHUNK, EMBED_DIM), jnp.float32),
        pltpu.VMEM((CHUNK, EMBED_DIM), jnp.float32),
        pltpu.VMEM((CHUNK, EMBED_DIM), jnp.float32),
        pltpu.VMEM((CHUNK, EMBED_DIM), jnp.float32),
        pltpu.VMEM((CHUNK + L,), jnp.float32),
        pltpu.VMEM((CHUNK + L,), jnp.float32),
        pltpu.SemaphoreType.DMA,
        pltpu.SemaphoreType.DMA,
        pltpu.SemaphoreType.DMA,
        pltpu.SemaphoreType.DMA,
        pltpu.SemaphoreType.DMA,
        pltpu.SemaphoreType.DMA,
    ],
)
def _score_edges(table, src_idx, dst_idx, out_hbm, *scratch):
    _sc_kernel_body(table, src_idx, dst_idx, out_hbm, *scratch)


def kernel(embedding, edge_index):
    pad = E_PAD - NUM_EDGES
    src = jnp.pad(edge_index[0], (0, pad))
    dst = jnp.pad(edge_index[1], (0, pad))
    scores = _score_edges(embedding, src, dst)
    return scores[:NUM_EDGES].reshape(NUM_EDGES, 1)


# speedup vs baseline: 1.5220x; 1.0098x over previous
import functools

import jax
import jax.numpy as jnp
from jax import lax
from jax.experimental import pallas as pl
from jax.experimental.pallas import tpu as pltpu
from jax.experimental.pallas import tpu_sc as plsc

NUM_NODES = 1100000
EMBED_DIM = 64
NUM_EDGES = 1000000
ROW = 2 * EMBED_DIM
TBN = 2048
Q = 550912

NC = 2
NS = 16
NW = NC * NS
L = 16

CHUNK = 224
SUB = 112
NSUB = CHUNK // SUB
NCH0 = 118
NCH1 = 162
B0 = NCH0 * CHUNK
B1 = NCH1 * CHUNK
BP = B0 + B1
E_PAD = BP * NS
GROUPS = CHUNK // L


def _sc_kernel_body(table, src_idx, dst_idx, out_hbm,
                    idx_s0, idx_d0, idx_s1, idx_d1,
                    half_s0, half_d0, half_s1, half_d1,
                    par_s0, par_d0, par_s1, par_d1,
                    rows_s0, rows_d0, rows_s1, rows_d1,
                    obuf0, obuf1, sem0, sem1, isem0, isem1, osem0, osem1):
    cid = lax.axis_index("c")
    e0 = lax.axis_index("s") * BP + cid * B0
    nch = jnp.where(cid == 0, NCH0, NCH1)
    iota = lax.iota(jnp.int32, L)
    last_lane = iota == (L - 1)

    def idx_copy(c, idx_s, idx_d, isem):
        off = e0 + c * CHUNK
        pltpu.async_copy(src_idx.at[pl.ds(off, CHUNK)], idx_s, isem)
        pltpu.async_copy(dst_idx.at[pl.ds(off, CHUNK)], idx_d, isem)

    def idx_wait(idx_s, idx_d, isem):
        pltpu.make_async_copy(src_idx.at[pl.ds(0, CHUNK)], idx_s, isem).wait()
        pltpu.make_async_copy(dst_idx.at[pl.ds(0, CHUNK)], idx_d, isem).wait()

    def split_idx(idx, half, par):
        def body(v, _):
            b = v * L
            x = idx[pl.ds(b, L)]
            hi = (x >= Q).astype(jnp.int32)
            half[pl.ds(b, L)] = x - hi * Q
            par[pl.ds(b, L)] = hi * EMBED_DIM
            return ()
        lax.fori_loop(0, CHUNK // L, body, (), unroll=True)

    def fire(half_s, half_d, rows_s, rows_d, sem):
        for j in range(NSUB):
            pltpu.async_copy(
                table.at[half_s.at[pl.ds(j * SUB, SUB)]],
                rows_s.at[pl.ds(j * SUB, SUB)], sem)
            pltpu.async_copy(
                table.at[half_d.at[pl.ds(j * SUB, SUB)]],
                rows_d.at[pl.ds(j * SUB, SUB)], sem)

    def drain(rows_s, rows_d, sem):
        pltpu.make_async_copy(table.at[pl.ds(0, CHUNK)], rows_s, sem).wait()
        pltpu.make_async_copy(table.at[pl.ds(0, CHUNK)], rows_d, sem).wait()

    def compute(par_s, par_d, rows_s, rows_d, obuf):
        def group_body(g, _):
            eb = g * L
            pvs = par_s[pl.ds(eb, L)]
            pvd = par_d[pl.ds(eb, L)]
            for e in range(L):
                er = eb + e
                bs = pvs[e]
                bd = pvd[e]
                p = rows_s[er, pl.ds(bs, L)] * rows_d[er, pl.ds(bd, L)]
                for k in range(1, EMBED_DIM // L):
                    p = p + (rows_s[er, pl.ds(bs + k * L, L)]
                             * rows_d[er, pl.ds(bd + k * L, L)])
                cs = plsc.cumsum(p)
                plsc.store_compressed(obuf.at[pl.ds(er, L)], cs, mask=last_lane)
            return ()

        lax.fori_loop(0, GROUPS, group_body, (), unroll=False)

    bufs = (
        (idx_s0, idx_d0, half_s0, half_d0, par_s0, par_d0,
         rows_s0, rows_d0, obuf0, sem0, isem0, osem0),
        (idx_s1, idx_d1, half_s1, half_d1, par_s1, par_d1,
         rows_s1, rows_d1, obuf1, sem1, isem1, osem1),
    )

    for b in range(2):
        (idx_s, idx_d, half_s, half_d, par_s, par_d,
         rows_s, rows_d, obuf, sem, isem, osem) = bufs[b]
        idx_copy(b, idx_s, idx_d, isem)
        idx_wait(idx_s, idx_d, isem)
        split_idx(idx_s, half_s, par_s)
        split_idx(idx_d, half_d, par_d)
        fire(half_s, half_d, rows_s, rows_d, sem)

    def chunk_pair(i2, _):
        for b in range(2):
            c = i2 * 2 + b
            (idx_s, idx_d, half_s, half_d, par_s, par_d,
             rows_s, rows_d, obuf, sem, isem, osem) = bufs[b]

            @pl.when(c >= 2)
            def _():
                pltpu.make_async_copy(
                    obuf.at[pl.ds(0, CHUNK)],
                    out_hbm.at[pl.ds(0, CHUNK)], osem).wait()

            drain(rows_s, rows_d, sem)

            @pl.when(c + 2 < nch)
            def _():
                idx_copy(c + 2, idx_s, idx_d, isem)

            compute(par_s, par_d, rows_s, rows_d, obuf)
            pltpu.async_copy(obuf.at[pl.ds(0, CHUNK)],
                             out_hbm.at[pl.ds(e0 + c * CHUNK, CHUNK)], osem)

            @pl.when(c + 2 < nch)
            def _():
                idx_wait(idx_s, idx_d, isem)
                split_idx(idx_s, half_s, par_s)
                split_idx(idx_d, half_d, par_d)
                fire(half_s, half_d, rows_s, rows_d, sem)
        return ()

    lax.fori_loop(0, nch // 2, chunk_pair, (), unroll=False)

    for b in range(2):
        obuf, osem = bufs[b][8], bufs[b][11]
        pltpu.make_async_copy(obuf.at[pl.ds(0, CHUNK)],
                              out_hbm.at[pl.ds(0, CHUNK)], osem).wait()


@functools.partial(
    pl.kernel,
    mesh=plsc.VectorSubcoreMesh(core_axis_name="c", subcore_axis_name="s"),
    out_type=jax.ShapeDtypeStruct((E_PAD,), jnp.float32),
    compiler_params=pltpu.CompilerParams(
        needs_layout_passes=False, use_tc_tiling_on_sc=True),
    scratch_types=[
        pltpu.VMEM((CHUNK,), jnp.int32),
        pltpu.VMEM((CHUNK,), jnp.int32),
        pltpu.VMEM((CHUNK,), jnp.int32),
        pltpu.VMEM((CHUNK,), jnp.int32),
        pltpu.VMEM((CHUNK,), jnp.int32),
        pltpu.VMEM((CHUNK,), jnp.int32),
        pltpu.VMEM((CHUNK,), jnp.int32),
        pltpu.VMEM((CHUNK,), jnp.int32),
        pltpu.VMEM((CHUNK,), jnp.int32),
        pltpu.VMEM((CHUNK,), jnp.int32),
        pltpu.VMEM((CHUNK,), jnp.int32),
        pltpu.VMEM((CHUNK,), jnp.int32),
        pltpu.VMEM((CHUNK, ROW), jnp.float32),
        pltpu.VMEM((CHUNK, ROW), jnp.float32),
        pltpu.VMEM((CHUNK, ROW), jnp.float32),
        pltpu.VMEM((CHUNK, ROW), jnp.float32),
        pltpu.VMEM((CHUNK + L,), jnp.float32),
        pltpu.VMEM((CHUNK + L,), jnp.float32),
        pltpu.SemaphoreType.DMA,
        pltpu.SemaphoreType.DMA,
        pltpu.SemaphoreType.DMA,
        pltpu.SemaphoreType.DMA,
        pltpu.SemaphoreType.DMA,
        pltpu.SemaphoreType.DMA,
    ],
)
def _score_edges(table, src_idx, dst_idx, out_hbm, *scratch):
    _sc_kernel_body(table, src_idx, dst_idx, out_hbm, *scratch)


TGRID = Q // TBN


def _pack_body(a_ref, b_ref, out_ref):
    ident = jax.lax.broadcasted_iota(jnp.int32, (EMBED_DIM, EMBED_DIM), 0)
    ident = (ident == jax.lax.broadcasted_iota(
        jnp.int32, (EMBED_DIM, EMBED_DIM), 1)).astype(jnp.float32)
    xta = jax.lax.dot_general(
        a_ref[...], ident, (((0,), (0,)), ((), ())),
        preferred_element_type=jnp.float32)
    xtb = jax.lax.dot_general(
        b_ref[...], ident, (((0,), (0,)), ((), ())),
        preferred_element_type=jnp.float32)
    out_ref[:, 0:EMBED_DIM] = xta
    out_ref[:, EMBED_DIM:ROW] = xtb


_pack_pairs = pl.pallas_call(
    _pack_body,
    grid=(TGRID,),
    in_specs=[
        pl.BlockSpec((EMBED_DIM, TBN), lambda j: (0, j)),
        pl.BlockSpec((EMBED_DIM, TBN), lambda j: (0, j + TGRID)),
    ],
    out_specs=pl.BlockSpec((TBN, ROW), lambda j: (j, 0)),
    out_shape=jax.ShapeDtypeStruct((Q, ROW), jnp.float32),
)


def kernel(embedding, edge_index):
    pad = E_PAD - NUM_EDGES
    src = jnp.pad(edge_index[0], (0, pad))
    dst = jnp.pad(edge_index[1], (0, pad))
    emb_t = embedding.T
    pairs = _pack_pairs(emb_t, emb_t)
    scores = _score_edges(pairs, src, dst)
    return scores[:NUM_EDGES].reshape(NUM_EDGES, 1)


# speedup vs baseline: 1.6756x; 1.1009x over previous
import functools

import jax
import jax.numpy as jnp
from jax import lax
from jax.experimental import pallas as pl
from jax.experimental.pallas import tpu as pltpu
from jax.experimental.pallas import tpu_sc as plsc

NUM_NODES = 1100000
EMBED_DIM = 64
NUM_EDGES = 1000000
ROW = 2 * EMBED_DIM
TBN = 2048
Q = 550912

NC = 2
NS = 16
NW = NC * NS
L = 16

CHUNK = 224
SUB = 112
NSUB = CHUNK // SUB
NCH0 = 162
NCH1 = 118
B0 = NCH0 * CHUNK
B1 = NCH1 * CHUNK
BP = B0 + B1
E_PAD = BP * NS
GROUPS = CHUNK // L


def _sc_kernel_body(table, src_idx, dst_idx, out_hbm,
                    idx_s0, idx_d0, idx_s1, idx_d1,
                    half_s0, half_d0, half_s1, half_d1,
                    par_s0, par_d0, par_s1, par_d1,
                    rows_s0, rows_d0, rows_s1, rows_d1,
                    obuf0, obuf1, sem0, sem1, isem0, isem1, osem0, osem1):
    cid = lax.axis_index("c")
    e0 = lax.axis_index("s") * BP + cid * B0
    nch = jnp.where(cid == 0, NCH0, NCH1)
    iota = lax.iota(jnp.int32, L)
    last_lane = iota == (L - 1)

    def idx_copy(c, idx_s, idx_d, isem):
        off = e0 + c * CHUNK
        pltpu.async_copy(src_idx.at[pl.ds(off, CHUNK)], idx_s, isem)
        pltpu.async_copy(dst_idx.at[pl.ds(off, CHUNK)], idx_d, isem)

    def idx_wait(idx_s, idx_d, isem):
        pltpu.make_async_copy(src_idx.at[pl.ds(0, CHUNK)], idx_s, isem).wait()
        pltpu.make_async_copy(dst_idx.at[pl.ds(0, CHUNK)], idx_d, isem).wait()

    def split_idx(idx, half, par):
        def body(v, _):
            b = v * L
            x = idx[pl.ds(b, L)]
            hi = (x >= Q).astype(jnp.int32)
            half[pl.ds(b, L)] = x - hi * Q
            par[pl.ds(b, L)] = hi * EMBED_DIM
            return ()
        lax.fori_loop(0, CHUNK // L, body, (), unroll=True)

    def fire(half_s, half_d, rows_s, rows_d, sem):
        for j in range(NSUB):
            pltpu.async_copy(
                table.at[half_s.at[pl.ds(j * SUB, SUB)]],
                rows_s.at[pl.ds(j * SUB, SUB)], sem)
            pltpu.async_copy(
                table.at[half_d.at[pl.ds(j * SUB, SUB)]],
                rows_d.at[pl.ds(j * SUB, SUB)], sem)

    def drain(rows_s, rows_d, sem):
        pltpu.make_async_copy(table.at[pl.ds(0, CHUNK)], rows_s, sem).wait()
        pltpu.make_async_copy(table.at[pl.ds(0, CHUNK)], rows_d, sem).wait()

    def compute(par_s, par_d, rows_s, rows_d, obuf):
        def group_body(g, _):
            eb = g * L
            pvs = par_s[pl.ds(eb, L)]
            pvd = par_d[pl.ds(eb, L)]
            for e in range(L):
                er = eb + e
                bs = pvs[e]
                bd = pvd[e]
                p = rows_s[er, pl.ds(bs, L)] * rows_d[er, pl.ds(bd, L)]
                for k in range(1, EMBED_DIM // L):
                    p = p + (rows_s[er, pl.ds(bs + k * L, L)]
                             * rows_d[er, pl.ds(bd + k * L, L)])
                cs = plsc.cumsum(p)
                plsc.store_compressed(obuf.at[pl.ds(er, L)], cs, mask=last_lane)
            return ()

        lax.fori_loop(0, GROUPS, group_body, (), unroll=False)

    bufs = (
        (idx_s0, idx_d0, half_s0, half_d0, par_s0, par_d0,
         rows_s0, rows_d0, obuf0, sem0, isem0, osem0),
        (idx_s1, idx_d1, half_s1, half_d1, par_s1, par_d1,
         rows_s1, rows_d1, obuf1, sem1, isem1, osem1),
    )

    for b in range(2):
        (idx_s, idx_d, half_s, half_d, par_s, par_d,
         rows_s, rows_d, obuf, sem, isem, osem) = bufs[b]
        idx_copy(b, idx_s, idx_d, isem)
        idx_wait(idx_s, idx_d, isem)
        split_idx(idx_s, half_s, par_s)
        split_idx(idx_d, half_d, par_d)
        fire(half_s, half_d, rows_s, rows_d, sem)

    def chunk_pair(i2, _):
        for b in range(2):
            c = i2 * 2 + b
            (idx_s, idx_d, half_s, half_d, par_s, par_d,
             rows_s, rows_d, obuf, sem, isem, osem) = bufs[b]

            @pl.when(c >= 2)
            def _():
                pltpu.make_async_copy(
                    obuf.at[pl.ds(0, CHUNK)],
                    out_hbm.at[pl.ds(0, CHUNK)], osem).wait()

            drain(rows_s, rows_d, sem)

            @pl.when(c + 2 < nch)
            def _():
                idx_copy(c + 2, idx_s, idx_d, isem)

            compute(par_s, par_d, rows_s, rows_d, obuf)
            pltpu.async_copy(obuf.at[pl.ds(0, CHUNK)],
                             out_hbm.at[pl.ds(e0 + c * CHUNK, CHUNK)], osem)

            @pl.when(c + 2 < nch)
            def _():
                idx_wait(idx_s, idx_d, isem)
                split_idx(idx_s, half_s, par_s)
                split_idx(idx_d, half_d, par_d)
                fire(half_s, half_d, rows_s, rows_d, sem)
        return ()

    lax.fori_loop(0, nch // 2, chunk_pair, (), unroll=False)

    for b in range(2):
        obuf, osem = bufs[b][8], bufs[b][11]
        pltpu.make_async_copy(obuf.at[pl.ds(0, CHUNK)],
                              out_hbm.at[pl.ds(0, CHUNK)], osem).wait()


@functools.partial(
    pl.kernel,
    mesh=plsc.VectorSubcoreMesh(core_axis_name="c", subcore_axis_name="s"),
    out_type=jax.ShapeDtypeStruct((E_PAD,), jnp.float32),
    compiler_params=pltpu.CompilerParams(
        needs_layout_passes=False, use_tc_tiling_on_sc=True),
    scratch_types=[
        pltpu.VMEM((CHUNK,), jnp.int32),
        pltpu.VMEM((CHUNK,), jnp.int32),
        pltpu.VMEM((CHUNK,), jnp.int32),
        pltpu.VMEM((CHUNK,), jnp.int32),
        pltpu.VMEM((CHUNK,), jnp.int32),
        pltpu.VMEM((CHUNK,), jnp.int32),
        pltpu.VMEM((CHUNK,), jnp.int32),
        pltpu.VMEM((CHUNK,), jnp.int32),
        pltpu.VMEM((CHUNK,), jnp.int32),
        pltpu.VMEM((CHUNK,), jnp.int32),
        pltpu.VMEM((CHUNK,), jnp.int32),
        pltpu.VMEM((CHUNK,), jnp.int32),
        pltpu.VMEM((CHUNK, ROW), jnp.float32),
        pltpu.VMEM((CHUNK, ROW), jnp.float32),
        pltpu.VMEM((CHUNK, ROW), jnp.float32),
        pltpu.VMEM((CHUNK, ROW), jnp.float32),
        pltpu.VMEM((CHUNK + L,), jnp.float32),
        pltpu.VMEM((CHUNK + L,), jnp.float32),
        pltpu.SemaphoreType.DMA,
        pltpu.SemaphoreType.DMA,
        pltpu.SemaphoreType.DMA,
        pltpu.SemaphoreType.DMA,
        pltpu.SemaphoreType.DMA,
        pltpu.SemaphoreType.DMA,
    ],
)
def _score_edges(table, src_idx, dst_idx, out_hbm, *scratch):
    _sc_kernel_body(table, src_idx, dst_idx, out_hbm, *scratch)


TGRID = Q // TBN


def _pack_body(a_ref, b_ref, out_ref):
    ident = jax.lax.broadcasted_iota(jnp.int32, (EMBED_DIM, EMBED_DIM), 0)
    ident = (ident == jax.lax.broadcasted_iota(
        jnp.int32, (EMBED_DIM, EMBED_DIM), 1)).astype(jnp.float32)
    xta = jax.lax.dot_general(
        a_ref[...], ident, (((0,), (0,)), ((), ())),
        preferred_element_type=jnp.float32)
    xtb = jax.lax.dot_general(
        b_ref[...], ident, (((0,), (0,)), ((), ())),
        preferred_element_type=jnp.float32)
    out_ref[:, 0:EMBED_DIM] = xta
    out_ref[:, EMBED_DIM:ROW] = xtb


_pack_pairs = pl.pallas_call(
    _pack_body,
    grid=(TGRID,),
    in_specs=[
        pl.BlockSpec((EMBED_DIM, TBN), lambda j: (0, j)),
        pl.BlockSpec((EMBED_DIM, TBN), lambda j: (0, j + TGRID)),
    ],
    out_specs=pl.BlockSpec((TBN, ROW), lambda j: (j, 0)),
    out_shape=jax.ShapeDtypeStruct((Q, ROW), jnp.float32),
)


def kernel(embedding, edge_index):
    pad = E_PAD - NUM_EDGES
    src = jnp.pad(edge_index[0], (0, pad))
    dst = jnp.pad(edge_index[1], (0, pad))
    emb_t = embedding.T
    pairs = _pack_pairs(emb_t, emb_t)
    scores = _score_edges(pairs, src, dst)
    return scores[:NUM_EDGES].reshape(NUM_EDGES, 1)


# speedup vs baseline: 1.7046x; 1.0174x over previous
import functools

import jax
import jax.numpy as jnp
from jax import lax
from jax.experimental import pallas as pl
from jax.experimental.pallas import tpu as pltpu
from jax.experimental.pallas import tpu_sc as plsc

NUM_NODES = 1100000
EMBED_DIM = 64
NUM_EDGES = 1000000
ROW = 2 * EMBED_DIM
TBN = 2048
Q = 550912

NC = 2
NS = 16
NW = NC * NS
L = 16

CHUNK = 224
SUB = 112
NSUB = CHUNK // SUB
NCH0 = 170
NCH1 = 110
B0 = NCH0 * CHUNK
B1 = NCH1 * CHUNK
BP = B0 + B1
E_PAD = BP * NS
GROUPS = CHUNK // L


def _sc_kernel_body(table, src_idx, dst_idx, out_hbm,
                    idx_s0, idx_d0, idx_s1, idx_d1,
                    half_s0, half_d0, half_s1, half_d1,
                    par_s0, par_d0, par_s1, par_d1,
                    rows_s0, rows_d0, rows_s1, rows_d1,
                    obuf0, obuf1, sem0, sem1, isem0, isem1, osem0, osem1):
    cid = lax.axis_index("c")
    e0 = lax.axis_index("s") * BP + cid * B0
    nch = jnp.where(cid == 0, NCH0, NCH1)
    iota = lax.iota(jnp.int32, L)
    last_lane = iota == (L - 1)

    def idx_copy(c, idx_s, idx_d, isem):
        off = e0 + c * CHUNK
        pltpu.async_copy(src_idx.at[pl.ds(off, CHUNK)], idx_s, isem)
        pltpu.async_copy(dst_idx.at[pl.ds(off, CHUNK)], idx_d, isem)

    def idx_wait(idx_s, idx_d, isem):
        pltpu.make_async_copy(src_idx.at[pl.ds(0, CHUNK)], idx_s, isem).wait()
        pltpu.make_async_copy(dst_idx.at[pl.ds(0, CHUNK)], idx_d, isem).wait()

    def split_idx(idx, half, par):
        def body(v, _):
            b = v * L
            x = idx[pl.ds(b, L)]
            hi = (x >= Q).astype(jnp.int32)
            half[pl.ds(b, L)] = x - hi * Q
            par[pl.ds(b, L)] = hi * EMBED_DIM
            return ()
        lax.fori_loop(0, CHUNK // L, body, (), unroll=True)

    def fire(half_s, half_d, rows_s, rows_d, sem):
        for j in range(NSUB):
            pltpu.async_copy(
                table.at[half_s.at[pl.ds(j * SUB, SUB)]],
                rows_s.at[pl.ds(j * SUB, SUB)], sem)
            pltpu.async_copy(
                table.at[half_d.at[pl.ds(j * SUB, SUB)]],
                rows_d.at[pl.ds(j * SUB, SUB)], sem)

    def drain(rows_s, rows_d, sem):
        pltpu.make_async_copy(table.at[pl.ds(0, CHUNK)], rows_s, sem).wait()
        pltpu.make_async_copy(table.at[pl.ds(0, CHUNK)], rows_d, sem).wait()

    def compute(par_s, par_d, rows_s, rows_d, obuf):
        def group_body(g, _):
            eb = g * L
            pvs = par_s[pl.ds(eb, L)]
            pvd = par_d[pl.ds(eb, L)]
            for e in range(L):
                er = eb + e
                bs = pvs[e]
                bd = pvd[e]
                p = rows_s[er, pl.ds(bs, L)] * rows_d[er, pl.ds(bd, L)]
                for k in range(1, EMBED_DIM // L):
                    p = p + (rows_s[er, pl.ds(bs + k * L, L)]
                             * rows_d[er, pl.ds(bd + k * L, L)])
                cs = plsc.cumsum(p)
                plsc.store_compressed(obuf.at[pl.ds(er, L)], cs, mask=last_lane)
            return ()

        lax.fori_loop(0, GROUPS, group_body, (), unroll=False)

    bufs = (
        (idx_s0, idx_d0, half_s0, half_d0, par_s0, par_d0,
         rows_s0, rows_d0, obuf0, sem0, isem0, osem0),
        (idx_s1, idx_d1, half_s1, half_d1, par_s1, par_d1,
         rows_s1, rows_d1, obuf1, sem1, isem1, osem1),
    )

    for b in range(2):
        (idx_s, idx_d, half_s, half_d, par_s, par_d,
         rows_s, rows_d, obuf, sem, isem, osem) = bufs[b]
        idx_copy(b, idx_s, idx_d, isem)
        idx_wait(idx_s, idx_d, isem)
        split_idx(idx_s, half_s, par_s)
        split_idx(idx_d, half_d, par_d)
        fire(half_s, half_d, rows_s, rows_d, sem)

    def chunk_pair(i2, _):
        for b in range(2):
            c = i2 * 2 + b
            (idx_s, idx_d, half_s, half_d, par_s, par_d,
             rows_s, rows_d, obuf, sem, isem, osem) = bufs[b]

            @pl.when(c >= 2)
            def _():
                pltpu.make_async_copy(
                    obuf.at[pl.ds(0, CHUNK)],
                    out_hbm.at[pl.ds(0, CHUNK)], osem).wait()

            drain(rows_s, rows_d, sem)

            @pl.when(c + 2 < nch)
            def _():
                idx_copy(c + 2, idx_s, idx_d, isem)

            compute(par_s, par_d, rows_s, rows_d, obuf)
            pltpu.async_copy(obuf.at[pl.ds(0, CHUNK)],
                             out_hbm.at[pl.ds(e0 + c * CHUNK, CHUNK)], osem)

            @pl.when(c + 2 < nch)
            def _():
                idx_wait(idx_s, idx_d, isem)
                split_idx(idx_s, half_s, par_s)
                split_idx(idx_d, half_d, par_d)
                fire(half_s, half_d, rows_s, rows_d, sem)
        return ()

    lax.fori_loop(0, nch // 2, chunk_pair, (), unroll=False)

    for b in range(2):
        obuf, osem = bufs[b][8], bufs[b][11]
        pltpu.make_async_copy(obuf.at[pl.ds(0, CHUNK)],
                              out_hbm.at[pl.ds(0, CHUNK)], osem).wait()


@functools.partial(
    pl.kernel,
    mesh=plsc.VectorSubcoreMesh(core_axis_name="c", subcore_axis_name="s"),
    out_type=jax.ShapeDtypeStruct((E_PAD,), jnp.float32),
    compiler_params=pltpu.CompilerParams(
        needs_layout_passes=False, use_tc_tiling_on_sc=True),
    scratch_types=[
        pltpu.VMEM((CHUNK,), jnp.int32),
        pltpu.VMEM((CHUNK,), jnp.int32),
        pltpu.VMEM((CHUNK,), jnp.int32),
        pltpu.VMEM((CHUNK,), jnp.int32),
        pltpu.VMEM((CHUNK,), jnp.int32),
        pltpu.VMEM((CHUNK,), jnp.int32),
        pltpu.VMEM((CHUNK,), jnp.int32),
        pltpu.VMEM((CHUNK,), jnp.int32),
        pltpu.VMEM((CHUNK,), jnp.int32),
        pltpu.VMEM((CHUNK,), jnp.int32),
        pltpu.VMEM((CHUNK,), jnp.int32),
        pltpu.VMEM((CHUNK,), jnp.int32),
        pltpu.VMEM((CHUNK, ROW), jnp.float32),
        pltpu.VMEM((CHUNK, ROW), jnp.float32),
        pltpu.VMEM((CHUNK, ROW), jnp.float32),
        pltpu.VMEM((CHUNK, ROW), jnp.float32),
        pltpu.VMEM((CHUNK + L,), jnp.float32),
        pltpu.VMEM((CHUNK + L,), jnp.float32),
        pltpu.SemaphoreType.DMA,
        pltpu.SemaphoreType.DMA,
        pltpu.SemaphoreType.DMA,
        pltpu.SemaphoreType.DMA,
        pltpu.SemaphoreType.DMA,
        pltpu.SemaphoreType.DMA,
    ],
)
def _score_edges(table, src_idx, dst_idx, out_hbm, *scratch):
    _sc_kernel_body(table, src_idx, dst_idx, out_hbm, *scratch)


TGRID = Q // TBN


def _pack_body(a_ref, b_ref, out_ref):
    ident = jax.lax.broadcasted_iota(jnp.int32, (EMBED_DIM, EMBED_DIM), 0)
    ident = (ident == jax.lax.broadcasted_iota(
        jnp.int32, (EMBED_DIM, EMBED_DIM), 1)).astype(jnp.float32)
    xta = jax.lax.dot_general(
        a_ref[...], ident, (((0,), (0,)), ((), ())),
        preferred_element_type=jnp.float32)
    xtb = jax.lax.dot_general(
        b_ref[...], ident, (((0,), (0,)), ((), ())),
        preferred_element_type=jnp.float32)
    out_ref[:, 0:EMBED_DIM] = xta
    out_ref[:, EMBED_DIM:ROW] = xtb


_pack_pairs = pl.pallas_call(
    _pack_body,
    grid=(TGRID,),
    in_specs=[
        pl.BlockSpec((EMBED_DIM, TBN), lambda j: (0, j)),
        pl.BlockSpec((EMBED_DIM, TBN), lambda j: (0, j + TGRID)),
    ],
    out_specs=pl.BlockSpec((TBN, ROW), lambda j: (j, 0)),
    out_shape=jax.ShapeDtypeStruct((Q, ROW), jnp.float32),
)


def kernel(embedding, edge_index):
    pad = E_PAD - NUM_EDGES
    src = jnp.pad(edge_index[0], (0, pad))
    dst = jnp.pad(edge_index[1], (0, pad))
    emb_t = embedding.T
    pairs = _pack_pairs(emb_t, emb_t)
    scores = _score_edges(pairs, src, dst)
    return scores[:NUM_EDGES].reshape(NUM_EDGES, 1)
